# baseline (speedup 1.0000x reference)
import functools

import jax
import jax.numpy as jnp
from jax import lax
from jax.experimental import pallas as pl
from jax.experimental.pallas import tpu as pltpu
from jax.experimental.pallas import tpu_sc as plsc

_MAXLEN = 8192
_DIM = 1024

_info = plsc.get_sparse_core_info()
_NC = _info.num_cores
_NS = _info.num_subcores
_NW = _NC * _NS
_ROWS_PER_W = _MAXLEN // _NW

_C = 8
_NBUF = 14
_PF = 8
_NCHUNK = _ROWS_PER_W // _C

_mesh = plsc.VectorSubcoreMesh(core_axis_name="c", subcore_axis_name="s")


@functools.partial(
    pl.kernel,
    mesh=_mesh,
    out_type=jax.ShapeDtypeStruct((_MAXLEN, _DIM), jnp.float32),
    scratch_types=(
        [pltpu.VMEM((_NBUF, _C, _DIM), jnp.float32)]
        + [pltpu.SemaphoreType.DMA] * (2 * _NBUF)
    ),
)
def _copy_table(table_hbm, out_hbm, bufs, *sems):
    in_sems = sems[:_NBUF]
    out_sems = sems[_NBUF:]
    wid = lax.axis_index("s") * _NC + lax.axis_index("c")
    base = wid * _ROWS_PER_W

    def start_in(i):
        b = i % _NBUF
        return pltpu.async_copy(
            table_hbm.at[pl.ds(base + i * _C, _C)], bufs.at[b], in_sems[b]
        )

    def start_out(i):
        b = i % _NBUF
        return pltpu.async_copy(
            bufs.at[b], out_hbm.at[pl.ds(base + i * _C, _C)], out_sems[b]
        )

    h_in = [None] * _NCHUNK
    h_out = [None] * _NCHUNK
    out_waited = [False] * _NCHUNK
    for i in range(min(_PF, _NCHUNK)):
        h_in[i] = start_in(i)
    for i in range(_NCHUNK):
        h_in[i].wait()
        h_out[i] = start_out(i)
        nxt = i + _PF
        if nxt < _NCHUNK:
            old = nxt - _NBUF
            if old >= 0:
                h_out[old].wait()
                out_waited[old] = True
            h_in[nxt] = start_in(nxt)
    for i in range(_NCHUNK):
        if not out_waited[i]:
            h_out[i].wait()


def kernel(input, table):
    del input
    return _copy_table(table)


# speedup vs baseline: 1.0460x; 1.0460x over previous
import functools

import jax
import jax.numpy as jnp
from jax import lax
from jax.experimental import pallas as pl
from jax.experimental.pallas import tpu as pltpu
from jax.experimental.pallas import tpu_sc as plsc

_MAXLEN = 8192
_DIM = 1024

_info = plsc.get_sparse_core_info()
_NC = _info.num_cores
_NS = _info.num_subcores
_NW = _NC * _NS
_ROWS_PER_W = _MAXLEN // _NW

_C = 16
_NBUF = 7
_PF = 5
_NCHUNK = _ROWS_PER_W // _C

_mesh = plsc.VectorSubcoreMesh(core_axis_name="c", subcore_axis_name="s")


@functools.partial(
    pl.kernel,
    mesh=_mesh,
    out_type=jax.ShapeDtypeStruct((_MAXLEN, _DIM), jnp.float32),
    scratch_types=(
        [pltpu.VMEM((_NBUF, _C, _DIM), jnp.float32)]
        + [pltpu.SemaphoreType.DMA] * (2 * _NBUF)
    ),
)
def _copy_table(table_hbm, out_hbm, bufs, *sems):
    in_sems = sems[:_NBUF]
    out_sems = sems[_NBUF:]
    wid = lax.axis_index("s") * _NC + lax.axis_index("c")
    base = wid * _ROWS_PER_W

    def start_in(i):
        b = i % _NBUF
        return pltpu.async_copy(
            table_hbm.at[pl.ds(base + i * _C, _C)], bufs.at[b], in_sems[b]
        )

    def start_out(i):
        b = i % _NBUF
        return pltpu.async_copy(
            bufs.at[b], out_hbm.at[pl.ds(base + i * _C, _C)], out_sems[b]
        )

    h_in = [None] * _NCHUNK
    h_out = [None] * _NCHUNK
    out_waited = [False] * _NCHUNK
    for i in range(min(_PF, _NCHUNK)):
        h_in[i] = start_in(i)
    for i in range(_NCHUNK):
        h_in[i].wait()
        h_out[i] = start_out(i)
        nxt = i + _PF
        if nxt < _NCHUNK:
            old = nxt - _NBUF
            if old >= 0:
                h_out[old].wait()
                out_waited[old] = True
            h_in[nxt] = start_in(nxt)
    for i in range(_NCHUNK):
        if not out_waited[i]:
            h_out[i].wait()


def kernel(input, table):
    del input
    return _copy_table(table)
